# baseline (speedup 1.0000x reference)
import functools

import jax
import jax.numpy as jnp
from jax.experimental import pallas as pl
from jax.experimental.pallas import tpu as pltpu

C = 384
H = 224
W = 224
S = 16
NH = H // S
NW = W // S
K = NH * NW
KP = 224
PB = S * W
NEG = -1e30
INV_SQRT_C = float(1.0 / (C ** 0.5))


def _pool_kernel(x_ref, poolw_ref, out_ref):
    s = jnp.sum(x_ref[...], axis=1)
    out_ref[0] = jax.lax.dot_general(
        s, poolw_ref[...], (((1,), (0,)), ((), ())),
        preferred_element_type=jnp.float32)


def _ssn_kernel(pix_ref, cent_ref, wk_ref, wv_ref, ks_ref, vs_ref,
                num_acc, den_acc):
    bh = pl.program_id(0)
    pixb = pix_ref[...].reshape(C, PB)
    cent = cent_ref[...]
    dots = jax.lax.dot_general(
        cent, pixb, (((1,), (0,)), ((), ())),
        preferred_element_type=jnp.float32)
    cent_sq = jnp.sum(cent * cent, axis=1, keepdims=True)
    logits = 2.0 * dots - cent_sq
    ks2 = jax.lax.broadcasted_iota(jnp.int32, (KP, PB), 0)
    kh = ks2 // NW
    kw = ks2 % NW
    lp = jax.lax.broadcasted_iota(jnp.int32, (KP, PB), 1)
    bw = (lp % W) // S
    valid = ((jnp.abs(kh - bh) <= 1) & (jnp.abs(kw - bw) <= 1) & (ks2 < K))
    lm = jnp.where(valid, logits, NEG)
    m = jnp.max(lm, axis=0, keepdims=True)
    e = jnp.exp(lm - m)
    den = jnp.sum(e, axis=0, keepdims=True)
    aff = e / den
    contrib = jax.lax.dot_general(
        aff, pixb, (((1,), (1,)), ((), ())),
        preferred_element_type=jnp.float32)
    dcontrib = jnp.broadcast_to(
        jnp.sum(aff, axis=1, keepdims=True), (KP, 128))

    @pl.when(bh == 0)
    def _():
        num_acc[...] = contrib
        den_acc[...] = dcontrib

    @pl.when(bh > 0)
    def _():
        num_acc[...] += contrib
        den_acc[...] += dcontrib

    @pl.when(bh == NH - 1)
    def _():
        cent1 = num_acc[...] / (den_acc[...][:, :1] + 1e-16)
        ks_ref[...] = jnp.dot(cent1, wk_ref[...],
                              preferred_element_type=jnp.float32)
        vs_ref[...] = jnp.dot(cent1, wv_ref[...],
                              preferred_element_type=jnp.float32)


def _attn_kernel(pix_ref, wq_ref, wo_ref, ks_ref, vs_ref, y_ref):
    pixb = pix_ref[...].reshape(C, PB)
    qT = jax.lax.dot_general(
        wq_ref[...], pixb, (((0,), (0,)), ((), ())),
        preferred_element_type=jnp.float32)
    logits = jax.lax.dot_general(
        ks_ref[...], qT, (((1,), (0,)), ((), ())),
        preferred_element_type=jnp.float32) * INV_SQRT_C
    ks2 = jax.lax.broadcasted_iota(jnp.int32, (KP, PB), 0)
    lm = jnp.where(ks2 < K, logits, NEG)
    m = jnp.max(lm, axis=0, keepdims=True)
    e = jnp.exp(lm - m)
    attnT = e / jnp.sum(e, axis=0, keepdims=True)
    outT = jax.lax.dot_general(
        vs_ref[...], attnT, (((0,), (0,)), ((), ())),
        preferred_element_type=jnp.float32)
    projT = jax.lax.dot_general(
        wo_ref[...], outT, (((0,), (0,)), ((), ())),
        preferred_element_type=jnp.float32)
    y_ref[...] = (pixb + projT).reshape(C, S, W)


@functools.partial(jax.jit, static_argnames=("interpret",))
def kernel(x, Wq, Wk, Wv, Wo, interpret=False):
    x3 = x.reshape(C, H, W)
    poolw_np = (jnp.arange(W)[:, None] // S ==
                jnp.arange(NW)[None, :]).astype(jnp.float32) / (S * S)

    cent_rows = pl.pallas_call(
        _pool_kernel,
        grid=(NH,),
        in_specs=[
            pl.BlockSpec((C, S, W), lambda i: (0, i, 0)),
            pl.BlockSpec((W, NW), lambda i: (0, 0)),
        ],
        out_specs=pl.BlockSpec((1, C, NW), lambda i: (i, 0, 0)),
        out_shape=jax.ShapeDtypeStruct((NH, C, NW), jnp.float32),
        interpret=interpret,
    )(x3, poolw_np)
    cent0 = jnp.pad(cent_rows.transpose(0, 2, 1).reshape(K, C),
                    ((0, KP - K), (0, 0)))

    ks, vs = pl.pallas_call(
        _ssn_kernel,
        grid=(NH,),
        in_specs=[
            pl.BlockSpec((C, S, W), lambda i: (0, i, 0)),
            pl.BlockSpec((KP, C), lambda i: (0, 0)),
            pl.BlockSpec((C, C), lambda i: (0, 0)),
            pl.BlockSpec((C, C), lambda i: (0, 0)),
        ],
        out_specs=[
            pl.BlockSpec((KP, C), lambda i: (0, 0)),
            pl.BlockSpec((KP, C), lambda i: (0, 0)),
        ],
        out_shape=[
            jax.ShapeDtypeStruct((KP, C), jnp.float32),
            jax.ShapeDtypeStruct((KP, C), jnp.float32),
        ],
        scratch_shapes=[
            pltpu.VMEM((KP, C), jnp.float32),
            pltpu.VMEM((KP, 128), jnp.float32),
        ],
        interpret=interpret,
    )(x3, cent0, Wk, Wv)

    y = pl.pallas_call(
        _attn_kernel,
        grid=(NH,),
        in_specs=[
            pl.BlockSpec((C, S, W), lambda i: (0, i, 0)),
            pl.BlockSpec((C, C), lambda i: (0, 0)),
            pl.BlockSpec((C, C), lambda i: (0, 0)),
            pl.BlockSpec((KP, C), lambda i: (0, 0)),
            pl.BlockSpec((KP, C), lambda i: (0, 0)),
        ],
        out_specs=pl.BlockSpec((C, S, W), lambda i: (0, i, 0)),
        out_shape=jax.ShapeDtypeStruct((C, H, W), jnp.float32),
        compiler_params=pltpu.CompilerParams(
            dimension_semantics=("arbitrary",)),
        interpret=interpret,
    )(x3, Wq, Wo, ks, vs)

    return y.reshape(1, C, H, W)


# speedup vs baseline: 1.0014x; 1.0014x over previous
import functools

import jax
import jax.numpy as jnp
from jax.experimental import pallas as pl
from jax.experimental.pallas import tpu as pltpu

C = 384
H = 224
W = 224
S = 16
NH = H // S
NW = W // S
K = NH * NW
KP = 224
PB = S * W
NEG = -1e30
INV_SQRT_C = float(1.0 / (C ** 0.5))


def _pool_kernel(x_ref, poolw_ref, out_ref):
    s = jnp.sum(x_ref[...], axis=1)
    out_ref[0] = jax.lax.dot_general(
        s, poolw_ref[...], (((1,), (0,)), ((), ())),
        preferred_element_type=jnp.float32)


def _ssn_kernel(pix_ref, cent_ref, wk_ref, wv_ref, ks_ref, vs_ref,
                num_acc, den_acc):
    bh = pl.program_id(0)
    pixb = pix_ref[...].reshape(C, PB)
    pixb16 = pixb.astype(jnp.bfloat16)
    cent = cent_ref[...]
    dots = jax.lax.dot_general(
        cent.astype(jnp.bfloat16), pixb16, (((1,), (0,)), ((), ())),
        preferred_element_type=jnp.float32)
    cent_sq = jnp.sum(cent * cent, axis=1, keepdims=True)
    logits = 2.0 * dots - cent_sq
    ks2 = jax.lax.broadcasted_iota(jnp.int32, (KP, PB), 0)
    kh = ks2 // NW
    kw = ks2 % NW
    lp = jax.lax.broadcasted_iota(jnp.int32, (KP, PB), 1)
    bw = (lp % W) // S
    valid = ((jnp.abs(kh - bh) <= 1) & (jnp.abs(kw - bw) <= 1) & (ks2 < K))
    lm = jnp.where(valid, logits, NEG)
    m = jnp.max(lm, axis=0, keepdims=True)
    e = jnp.exp(lm - m)
    den = jnp.sum(e, axis=0, keepdims=True)
    aff = e / den
    contrib = jax.lax.dot_general(
        aff.astype(jnp.bfloat16), pixb16, (((1,), (1,)), ((), ())),
        preferred_element_type=jnp.float32)
    dcontrib = jnp.broadcast_to(
        jnp.sum(aff, axis=1, keepdims=True), (KP, 128))

    @pl.when(bh == 0)
    def _():
        num_acc[...] = contrib
        den_acc[...] = dcontrib

    @pl.when(bh > 0)
    def _():
        num_acc[...] += contrib
        den_acc[...] += dcontrib

    @pl.when(bh == NH - 1)
    def _():
        cent1 = (num_acc[...] /
                 (den_acc[...][:, :1] + 1e-16)).astype(jnp.bfloat16)
        ks_ref[...] = jnp.dot(cent1, wk_ref[...].astype(jnp.bfloat16),
                              preferred_element_type=jnp.float32)
        vs_ref[...] = jnp.dot(cent1, wv_ref[...].astype(jnp.bfloat16),
                              preferred_element_type=jnp.float32)


def _attn_kernel(pix_ref, wq_ref, wo_ref, ks_ref, vs_ref, y_ref):
    pixb = pix_ref[...].reshape(C, PB)
    qT = jax.lax.dot_general(
        wq_ref[...].astype(jnp.bfloat16), pixb.astype(jnp.bfloat16),
        (((0,), (0,)), ((), ())),
        preferred_element_type=jnp.float32)
    logits = jax.lax.dot_general(
        ks_ref[...].astype(jnp.bfloat16), qT.astype(jnp.bfloat16),
        (((1,), (0,)), ((), ())),
        preferred_element_type=jnp.float32) * INV_SQRT_C
    ks2 = jax.lax.broadcasted_iota(jnp.int32, (KP, PB), 0)
    lm = jnp.where(ks2 < K, logits, NEG)
    m = jnp.max(lm, axis=0, keepdims=True)
    e = jnp.exp(lm - m)
    attnT = e / jnp.sum(e, axis=0, keepdims=True)
    outT = jax.lax.dot_general(
        vs_ref[...].astype(jnp.bfloat16), attnT.astype(jnp.bfloat16),
        (((0,), (0,)), ((), ())),
        preferred_element_type=jnp.float32)
    projT = jax.lax.dot_general(
        wo_ref[...].astype(jnp.bfloat16), outT.astype(jnp.bfloat16),
        (((0,), (0,)), ((), ())),
        preferred_element_type=jnp.float32)
    y_ref[...] = (pixb + projT).reshape(C, S, W)


@functools.partial(jax.jit, static_argnames=("interpret",))
def kernel(x, Wq, Wk, Wv, Wo, interpret=False):
    x3 = x.reshape(C, H, W)
    poolw_np = (jnp.arange(W)[:, None] // S ==
                jnp.arange(NW)[None, :]).astype(jnp.float32) / (S * S)

    cent_rows = pl.pallas_call(
        _pool_kernel,
        grid=(NH,),
        in_specs=[
            pl.BlockSpec((C, S, W), lambda i: (0, i, 0)),
            pl.BlockSpec((W, NW), lambda i: (0, 0)),
        ],
        out_specs=pl.BlockSpec((1, C, NW), lambda i: (i, 0, 0)),
        out_shape=jax.ShapeDtypeStruct((NH, C, NW), jnp.float32),
        interpret=interpret,
    )(x3, poolw_np)
    cent0 = jnp.pad(cent_rows.transpose(0, 2, 1).reshape(K, C),
                    ((0, KP - K), (0, 0)))

    ks, vs = pl.pallas_call(
        _ssn_kernel,
        grid=(NH,),
        in_specs=[
            pl.BlockSpec((C, S, W), lambda i: (0, i, 0)),
            pl.BlockSpec((KP, C), lambda i: (0, 0)),
            pl.BlockSpec((C, C), lambda i: (0, 0)),
            pl.BlockSpec((C, C), lambda i: (0, 0)),
        ],
        out_specs=[
            pl.BlockSpec((KP, C), lambda i: (0, 0)),
            pl.BlockSpec((KP, C), lambda i: (0, 0)),
        ],
        out_shape=[
            jax.ShapeDtypeStruct((KP, C), jnp.float32),
            jax.ShapeDtypeStruct((KP, C), jnp.float32),
        ],
        scratch_shapes=[
            pltpu.VMEM((KP, C), jnp.float32),
            pltpu.VMEM((KP, 128), jnp.float32),
        ],
        interpret=interpret,
    )(x3, cent0, Wk, Wv)

    y = pl.pallas_call(
        _attn_kernel,
        grid=(NH,),
        in_specs=[
            pl.BlockSpec((C, S, W), lambda i: (0, i, 0)),
            pl.BlockSpec((C, C), lambda i: (0, 0)),
            pl.BlockSpec((C, C), lambda i: (0, 0)),
            pl.BlockSpec((KP, C), lambda i: (0, 0)),
            pl.BlockSpec((KP, C), lambda i: (0, 0)),
        ],
        out_specs=pl.BlockSpec((C, S, W), lambda i: (0, i, 0)),
        out_shape=jax.ShapeDtypeStruct((C, H, W), jnp.float32),
        compiler_params=pltpu.CompilerParams(
            dimension_semantics=("arbitrary",)),
        interpret=interpret,
    )(x3, Wq, Wo, ks, vs)

    return y.reshape(1, C, H, W)


# speedup vs baseline: 1.2320x; 1.2303x over previous
import functools

import jax
import jax.numpy as jnp
from jax.experimental import pallas as pl
from jax.experimental.pallas import tpu as pltpu

C = 384
H = 224
W = 224
S = 16
NH = H // S
NW = W // S
K = NH * NW
G = 16
KG = (NH + 2) * G
WIN = 3 * G
PB = S * W
NEG = -1e30
INV_SQRT_C = float(1.0 / (C ** 0.5))
F32 = jnp.float32
BF16 = jnp.bfloat16


def _ssn_kernel(x_ref, poolw_ref, mask_ref, wk_ref, wv_ref,
                pix16_ref, ks_ref, vs_ref,
                cent_scr, num_scr, den_scr, prev_scr):
    i = pl.program_id(0)

    @pl.when(i == 0)
    def _():
        cent_scr[pl.ds(0, G), :] = jnp.zeros((G, C), F32)
        cent_scr[pl.ds(KG - G, G), :] = jnp.zeros((G, C), F32)
        num_scr[...] = jnp.zeros((KG, C), F32)
        den_scr[...] = jnp.zeros((KG, 128), F32)

    @pl.when(i < NH)
    def _():
        xb = x_ref[...]
        rowsum = jnp.sum(xb, axis=1)
        poolT = jax.lax.dot_general(
            poolw_ref[...], rowsum, (((0,), (1,)), ((), ())),
            preferred_element_type=F32)
        cent_scr[pl.ds((i + 1) * G, G), :] = poolT

    @pl.when(i >= 1)
    def _():
        bh = i - 1
        centw = cent_scr[pl.ds(bh * G, WIN), :]
        prev = prev_scr[...]
        dots = jax.lax.dot_general(
            centw.astype(BF16), prev, (((1,), (0,)), ((), ())),
            preferred_element_type=F32)
        csq = jnp.sum(centw * centw, axis=1, keepdims=True)
        r = jax.lax.broadcasted_iota(jnp.int32, (WIN, 1), 0)
        kh = bh - 1 + r // G
        pen = jnp.where((kh >= 0) & (kh < NH), 0.0, -NEG)
        lm = 2.0 * dots - (csq + pen) + mask_ref[...]
        m = jnp.max(lm, axis=0, keepdims=True)
        e = jnp.exp(lm - m)
        den = jnp.sum(e, axis=0, keepdims=True)
        aff = e / den
        contrib = jax.lax.dot_general(
            aff.astype(BF16), prev, (((1,), (1,)), ((), ())),
            preferred_element_type=F32)
        num_scr[pl.ds(bh * G, WIN), :] += contrib
        den_scr[pl.ds(bh * G, WIN), :] += jnp.broadcast_to(
            jnp.sum(aff, axis=1, keepdims=True), (WIN, 128))

    @pl.when(i < NH)
    def _():
        pixm = x_ref[...].astype(BF16).reshape(C, PB)
        pix16_ref[0] = pixm
        prev_scr[...] = pixm

    @pl.when(i == NH)
    def _():
        cent1 = (num_scr[...] /
                 (den_scr[...][:, :1] + 1e-16)).astype(BF16)
        ks_ref[...] = jax.lax.dot_general(
            cent1, wk_ref[...].astype(BF16), (((1,), (0,)), ((), ())),
            preferred_element_type=F32).astype(BF16)
        vs_ref[...] = jax.lax.dot_general(
            cent1, wv_ref[...].astype(BF16), (((1,), (0,)), ((), ())),
            preferred_element_type=F32).astype(BF16)


def _attn_kernel(pix_ref, wq_ref, wo_ref, ks_ref, vs_ref, y_ref):
    pixj = pix_ref[0]
    qT = jax.lax.dot_general(
        wq_ref[...].astype(BF16), pixj, (((0,), (0,)), ((), ())),
        preferred_element_type=F32)
    logits = jax.lax.dot_general(
        ks_ref[...], qT.astype(BF16), (((1,), (0,)), ((), ())),
        preferred_element_type=F32) * INV_SQRT_C
    r = jax.lax.broadcasted_iota(jnp.int32, (KG, 1), 0)
    colmask = jnp.where((r >= G) & (r < KG - G) & (r % G < NW), 0.0, NEG)
    lm = logits + colmask
    m = jnp.max(lm, axis=0, keepdims=True)
    e = jnp.exp(lm - m)
    attnT = e / jnp.sum(e, axis=0, keepdims=True)
    outT = jax.lax.dot_general(
        vs_ref[...], attnT.astype(BF16), (((0,), (0,)), ((), ())),
        preferred_element_type=F32)
    projT = jax.lax.dot_general(
        wo_ref[...].astype(BF16), outT.astype(BF16), (((0,), (0,)), ((), ())),
        preferred_element_type=F32)
    y_ref[...] = (pixj.astype(F32) + projT).reshape(C, S, W)


@functools.partial(jax.jit, static_argnames=("interpret",))
def kernel(x, Wq, Wk, Wv, Wo, interpret=False):
    x3 = x.reshape(C, H, W)
    poolw = (jnp.arange(G)[:, None] ==
             jnp.arange(W)[None, :] // S).astype(F32) / (S * S)
    poolw = poolw.T
    rr = jnp.arange(WIN)[:, None]
    ll = jnp.arange(PB)[None, :]
    kw = rr % G
    bw = (ll % W) // S
    maskadd = jnp.where((jnp.abs(kw - bw) <= 1) & (kw < NW), 0.0, NEG
                        ).astype(F32)

    pix16, ks16, vs16 = pl.pallas_call(
        _ssn_kernel,
        grid=(NH + 1,),
        in_specs=[
            pl.BlockSpec((C, S, W), lambda i: (0, jnp.minimum(i, NH - 1), 0)),
            pl.BlockSpec((W, G), lambda i: (0, 0)),
            pl.BlockSpec((WIN, PB), lambda i: (0, 0)),
            pl.BlockSpec((C, C), lambda i: (0, 0)),
            pl.BlockSpec((C, C), lambda i: (0, 0)),
        ],
        out_specs=[
            pl.BlockSpec((1, C, PB), lambda i: (jnp.minimum(i, NH - 1), 0, 0)),
            pl.BlockSpec((KG, C), lambda i: (0, 0)),
            pl.BlockSpec((KG, C), lambda i: (0, 0)),
        ],
        out_shape=[
            jax.ShapeDtypeStruct((NH, C, PB), BF16),
            jax.ShapeDtypeStruct((KG, C), BF16),
            jax.ShapeDtypeStruct((KG, C), BF16),
        ],
        scratch_shapes=[
            pltpu.VMEM((KG, C), F32),
            pltpu.VMEM((KG, C), F32),
            pltpu.VMEM((KG, 128), F32),
            pltpu.VMEM((C, PB), BF16),
        ],
        interpret=interpret,
    )(x3, poolw, maskadd, Wk, Wv)

    y = pl.pallas_call(
        _attn_kernel,
        grid=(NH,),
        in_specs=[
            pl.BlockSpec((1, C, PB), lambda j: (j, 0, 0)),
            pl.BlockSpec((C, C), lambda j: (0, 0)),
            pl.BlockSpec((C, C), lambda j: (0, 0)),
            pl.BlockSpec((KG, C), lambda j: (0, 0)),
            pl.BlockSpec((KG, C), lambda j: (0, 0)),
        ],
        out_specs=pl.BlockSpec((C, S, W), lambda j: (0, j, 0)),
        out_shape=jax.ShapeDtypeStruct((C, H, W), F32),
        compiler_params=pltpu.CompilerParams(
            dimension_semantics=("arbitrary",)),
        interpret=interpret,
    )(pix16, Wq, Wo, ks16, vs16)

    return y.reshape(1, C, H, W)
